# baseline (speedup 1.0000x reference)
import functools

import jax
import jax.numpy as jnp
from jax import lax
from jax.experimental import pallas as pl
from jax.experimental.pallas import tpu as pltpu
from jax.experimental.pallas import tpu_sc as plsc

N = 10000
E = 320000
B = 16
H = 64

NC, NS = 2, 16
NW = NC * NS
CH = 128
HALF = E // 2
NCH = HALF // CH
CPW = 40
NPADC = CPW * NW
NPAD_N = 10240
NPT = NPAD_N // NS

HB = 3200
GRID_E = E // HB
GRID_H = HALF // HB
BLK_N = 2000
GRID_N = N // BLK_N

@functools.cache
def _sc_mesh():
    return plsc.VectorSubcoreMesh(core_axis_name="c", subcore_axis_name="s",
                                  num_cores=NC, num_subcores=NS)


def _dot(a, b):
    return jnp.dot(a, b, preferred_element_type=jnp.float32)


def _gather_body(pr, pc, rowp, colp, g2, idxr, idxc, balo, bblo, bahi, bbhi,
                 balo2, bblo2, bahi2, bbhi2, obufa, obufb,
                 sema, semb, semoa, semob):
    w = lax.axis_index("s") * NC + lax.axis_index("c")
    start = w * CPW
    n = jnp.minimum(CPW, NCH - start)
    pltpu.sync_copy(rowp.at[pl.ds(start, CPW)], idxr.at[pl.ds(0, CPW)])
    pltpu.sync_copy(rowp.at[pl.ds(NPADC + start, CPW)],
                    idxr.at[pl.ds(CPW, CPW)])
    pltpu.sync_copy(colp.at[pl.ds(start, CPW)], idxc.at[pl.ds(0, CPW)])
    pltpu.sync_copy(colp.at[pl.ds(NPADC + start, CPW)],
                    idxc.at[pl.ds(CPW, CPW)])

    def issue(j, bufs, sem_g):
        return [pltpu.async_copy(pr.at[idxr.at[j]], bufs[0], sem_g),
                pltpu.async_copy(pc.at[idxc.at[j]], bufs[1], sem_g),
                pltpu.async_copy(pr.at[idxr.at[CPW + j]], bufs[2], sem_g),
                pltpu.async_copy(pc.at[idxc.at[CPW + j]], bufs[3], sem_g)]

    def pack_and_store(j, bufs, sem_g, obuf, sem_o, first):
        pltpu.make_async_copy(pr.at[idxr.at[j]], bufs[0], sem_g).wait()
        pltpu.make_async_copy(pc.at[idxc.at[j]], bufs[1], sem_g).wait()
        pltpu.make_async_copy(pr.at[idxr.at[CPW + j]], bufs[2], sem_g).wait()
        pltpu.make_async_copy(pc.at[idxc.at[CPW + j]], bufs[3], sem_g).wait()

        @pl.when(jnp.logical_not(first))
        def _():
            pltpu.make_async_copy(obuf, g2.at[pl.ds(0, CH)], sem_o).wait()

        def packrow(p, carry2):
            for r in range(16):
                i = 16 * p + r
                for k in range(H // 16):
                    sl = pl.ds(k * 16, 16)
                    obuf[i, pl.ds(k * 16, 16)] = bufs[0][i, sl] + bufs[1][i, sl]
                    obuf[i, pl.ds(H + k * 16, 16)] = (bufs[2][i, sl]
                                                      + bufs[3][i, sl])
            return carry2

        lax.fori_loop(0, CH // 16, packrow, 0)
        pltpu.async_copy(obuf, g2.at[pl.ds((start + j) * CH, CH)], sem_o)

    seta = (balo, bblo, bahi, bbhi)
    setb = (balo2, bblo2, bahi2, bbhi2)

    issue(0, seta, sema)

    def pair(t, carry):
        j = 2 * t

        @pl.when(j + 1 < n)
        def _():
            issue(j + 1, setb, semb)

        pack_and_store(j, seta, sema, obufa, semoa, t == 0)

        @pl.when(j + 1 < n)
        def _():
            @pl.when(j + 2 < n)
            def _():
                issue(j + 2, seta, sema)

            pack_and_store(j + 1, setb, semb, obufb, semob, t == 0)

        return carry

    lax.fori_loop(0, (n + 1) // 2, pair, 0)
    pltpu.make_async_copy(obufa, g2.at[pl.ds(0, CH)], semoa).wait()
    pltpu.make_async_copy(obufb, g2.at[pl.ds(0, CH)], semob).wait()


@functools.cache
def _gather_kernel():
    return pl.kernel(
        _gather_body,
        out_type=jax.ShapeDtypeStruct((HALF, 128), jnp.float32),
        mesh=_sc_mesh(),
        compiler_params=pltpu.CompilerParams(use_tc_tiling_on_sc=False, needs_layout_passes=False),
        scratch_types=[
            pltpu.VMEM((2 * CPW, CH), jnp.int32),
            pltpu.VMEM((2 * CPW, CH), jnp.int32),
            pltpu.VMEM((CH, H), jnp.float32),
            pltpu.VMEM((CH, H), jnp.float32),
            pltpu.VMEM((CH, H), jnp.float32),
            pltpu.VMEM((CH, H), jnp.float32),
            pltpu.VMEM((CH, H), jnp.float32),
            pltpu.VMEM((CH, H), jnp.float32),
            pltpu.VMEM((CH, H), jnp.float32),
            pltpu.VMEM((CH, H), jnp.float32),
            pltpu.VMEM((CH, 128), jnp.float32),
            pltpu.VMEM((CH, 128), jnp.float32),
            pltpu.SemaphoreType.DMA,
            pltpu.SemaphoreType.DMA,
            pltpu.SemaphoreType.DMA,
            pltpu.SemaphoreType.DMA,
        ],
    )


def _gather_call(pr, pc, rowp, colp):
    return _gather_kernel()(pr, pc, rowp, colp)


def _scatter_body(eoTl, eoTh, colp, parts, idxc, dbufT, dbufT2, dbuf, dbuf2,
                  zbuf, shared, semil, semih, semal, semah):
    c_ax = lax.axis_index("c")
    s_ax = lax.axis_index("s")
    w = s_ax * NC + c_ax
    start = w * CPW
    n = jnp.minimum(CPW, NCH - start)
    pltpu.sync_copy(colp.at[pl.ds(start, CPW)], idxc.at[pl.ds(0, CPW)])
    pltpu.sync_copy(colp.at[pl.ds(NPADC + start, CPW)],
                    idxc.at[pl.ds(CPW, CPW)])

    def zrow(i, carry):
        zbuf[i, :] = jnp.zeros((16,), jnp.float32)
        return carry

    lax.fori_loop(0, NPT, zrow, 0)
    pltpu.sync_copy(zbuf, shared.at[pl.ds(s_ax * NPT, NPT)])
    plsc.subcore_barrier()

    lane = lax.broadcasted_iota(jnp.int32, (16,), 0)

    def transpose(dT, db):
        for g in range(CH // 16):
            idx_e = lane + (g * 16)
            for f in range(16):
                plsc.store_scatter(db, [idx_e, jnp.full((16,), f, jnp.int32)],
                                   dT[f, pl.ds(g * 16, 16)])

    def half(k, src, dT, db, sem_i, sem_a, idx_row, prefetch, first):
        pltpu.make_async_copy(src.at[:, pl.ds(0, CH)], dT, sem_i).wait()

        @pl.when(jnp.logical_not(first))
        def _():
            pltpu.make_async_copy(db, shared.at[idxc.at[0]], sem_a).wait()

        transpose(dT, db)
        pltpu.async_copy(db, shared.at[idx_row], sem_a, add=True)

        @pl.when(k + 1 < n)
        def _():
            prefetch(k + 1)

    def pre_lo(k):
        pltpu.async_copy(eoTl.at[:, pl.ds((start + k) * CH, CH)], dbufT, semil)

    def pre_hi(k):
        pltpu.async_copy(eoTh.at[:, pl.ds((start + k) * CH, CH)], dbufT2,
                         semih)

    pre_lo(0)
    pre_hi(0)

    def chunk(j, carry):
        half(j, eoTl, dbufT, dbuf, semil, semal, idxc.at[j], pre_lo, j == 0)
        half(j, eoTh, dbufT2, dbuf2, semih, semah, idxc.at[CPW + j], pre_hi,
             j == 0)
        return carry

    lax.fori_loop(0, n, chunk, 0)
    pltpu.make_async_copy(dbuf, shared.at[idxc.at[0]], semal).wait()
    pltpu.make_async_copy(dbuf2, shared.at[idxc.at[0]], semah).wait()
    plsc.subcore_barrier()
    pltpu.sync_copy(shared.at[pl.ds(s_ax * NPT, NPT)],
                    parts.at[c_ax, pl.ds(s_ax * NPT, NPT)])


@functools.cache
def _scatter_kernel():
    return pl.kernel(
        _scatter_body,
        out_type=jax.ShapeDtypeStruct((NC, NPAD_N, 16), jnp.float32),
        mesh=_sc_mesh(),
        compiler_params=pltpu.CompilerParams(use_tc_tiling_on_sc=False, needs_layout_passes=False),
        scratch_types=[
            pltpu.VMEM((2 * CPW, CH), jnp.int32),
            pltpu.VMEM((16, CH), jnp.float32),
            pltpu.VMEM((16, CH), jnp.float32),
            pltpu.VMEM((CH, 16), jnp.float32),
            pltpu.VMEM((CH, 16), jnp.float32),
            pltpu.VMEM((NPT, 16), jnp.float32),
            pltpu.VMEM_SHARED((NPAD_N, 16), jnp.float32),
            pltpu.SemaphoreType.DMA,
            pltpu.SemaphoreType.DMA,
            pltpu.SemaphoreType.DMA,
            pltpu.SemaphoreType.DMA,
        ],
    )


def _scatter_call(eoTl, eoTh, colp):
    return _scatter_kernel()(eoTl, eoTh, colp)


def _prep_body(x_ref, wxr, wxc, wnx, u_ref, weu, eb1_ref, wnu, nb1_ref,
               pr_ref, pc_ref, pn_ref, ue_ref, un_ref):
    xb = x_ref[...]
    pr_ref[...] = _dot(xb, wxr[...])
    pc_ref[...] = _dot(xb, wxc[...])
    pn_ref[...] = _dot(xb, wnx[...])
    ub = u_ref[...]
    ue_ref[...] = _dot(ub, weu[...]) + eb1_ref[...]
    un_ref[...] = _dot(ub, wnu[...]) + nb1_ref[...]


def _prep_call(x, wxr, wxc, wnx, u, weu, eb1, wnu, nb1):
    full = lambda s: pl.BlockSpec(s, lambda i: (0, 0))
    return pl.pallas_call(
        _prep_body,
        grid=(GRID_N,),
        in_specs=[
            pl.BlockSpec((BLK_N, 128), lambda i: (i, 0)),
            full((128, H)), full((128, H)), full((128, H)),
            full((B, 32)), full((32, H)), full((1, H)), full((32, H)),
            full((1, H)),
        ],
        out_specs=[
            pl.BlockSpec((BLK_N, H), lambda i: (i, 0)),
            pl.BlockSpec((BLK_N, H), lambda i: (i, 0)),
            pl.BlockSpec((BLK_N, H), lambda i: (i, 0)),
            full((B, H)), full((B, H)),
        ],
        out_shape=[
            jax.ShapeDtypeStruct((N, H), jnp.float32),
            jax.ShapeDtypeStruct((N, H), jnp.float32),
            jax.ShapeDtypeStruct((N, H), jnp.float32),
            jax.ShapeDtypeStruct((B, H), jnp.float32),
            jax.ShapeDtypeStruct((B, H), jnp.float32),
        ],
    )(x, wxr, wxc, wnx, u, weu, eb1, wnu, nb1)


def _ln_T(h2, eb2c, egc, ebetac):
    h2 = jnp.maximum(h2 + eb2c, 0.0)
    mu = jnp.mean(h2, axis=0, keepdims=True)
    d = h2 - mu
    var = jnp.mean(d * d, axis=0, keepdims=True)
    return d * lax.rsqrt(var + 1e-5) * egc + ebetac


def _edge_body(g2_ref, eaT2_ref, el_ref, eh_ref, uebd_ref, weabd, w2bd,
               eb2_ref, eg_ref, ebeta_ref, eoTl_ref, eoTh_ref, se_ref,
               ce_ref):
    i = pl.program_id(0)
    dg = lambda a, b, dims: lax.dot_general(a, b, (dims, ((), ())),
                                            preferred_element_type=jnp.float32)
    el = el_ref[0, 0, :][None, :]
    eh = eh_ref[0, 0, :][None, :]
    gid = lax.broadcasted_iota(jnp.int32, (B, HB), 0)
    ohl = (el == gid).astype(jnp.float32)
    ohh = (eh == gid).astype(jnp.float32)
    oh2 = jnp.concatenate([ohl, ohh], axis=0)
    a = dg(eaT2_ref[...], weabd[...], ((0,), (0,)))
    u = dg(oh2, uebd_ref[...], ((0,), (0,)))
    h = jnp.maximum(g2_ref[...] + a + u, 0.0)
    h2 = dg(w2bd[...], h, ((0,), (1,)))
    eoTl = _ln_T(h2[:16], eb2_ref[...], eg_ref[...], ebeta_ref[...])
    eoTh = _ln_T(h2[16:], eb2_ref[...], eg_ref[...], ebeta_ref[...])
    eoTl_ref[...] = eoTl
    eoTh_ref[...] = eoTh

    @pl.when(i == 0)
    def _():
        se_ref[...] = jnp.zeros_like(se_ref)
        ce_ref[...] = jnp.zeros_like(ce_ref)

    se_ref[...] += (dg(eoTl, ohl, ((1,), (1,)))
                    + dg(eoTh, ohh, ((1,), (1,))))
    cnt_col = (jnp.sum(ohl, axis=1, keepdims=True)
               + jnp.sum(ohh, axis=1, keepdims=True))
    eye = (lax.broadcasted_iota(jnp.int32, (B, B), 0)
           == lax.broadcasted_iota(jnp.int32, (B, B), 1)).astype(jnp.float32)
    cnt_row = dg(cnt_col, eye, ((0,), (0,)))
    ce_ref[...] += jnp.broadcast_to(cnt_row, (8, B))


def _edge_call(g2, eaT2, el3, eh3, uebd, weabd, w2bd, eb2c, egc, ebetac):
    full = lambda s: pl.BlockSpec(s, lambda i: (0,) * len(s))
    return pl.pallas_call(
        _edge_body,
        grid=(GRID_H,),
        in_specs=[
            pl.BlockSpec((HB, 128), lambda i: (i, 0)),
            pl.BlockSpec((32, HB), lambda i: (0, i)),
            pl.BlockSpec((1, 1, HB), lambda i: (i, 0, 0)),
            pl.BlockSpec((1, 1, HB), lambda i: (i, 0, 0)),
            full((2 * B, 128)), full((32, 128)), full((128, 32)),
            full((16, 1)), full((16, 1)), full((16, 1)),
        ],
        out_specs=[
            pl.BlockSpec((16, HB), lambda i: (0, i)),
            pl.BlockSpec((16, HB), lambda i: (0, i)),
            full((16, B)), full((8, B)),
        ],
        out_shape=[
            jax.ShapeDtypeStruct((16, HALF), jnp.float32),
            jax.ShapeDtypeStruct((16, HALF), jnp.float32),
            jax.ShapeDtypeStruct((16, B), jnp.float32),
            jax.ShapeDtypeStruct((8, B), jnp.float32),
        ],
    )(g2, eaT2, el3, eh3, uebd, weabd, w2bd, eb2c, egc, ebetac)


def _node_body(pn_ref, s0_ref, s1_ref, vidx_ref, un_ref, wne, nw2, nb2_ref,
               ng_ref, nbeta_ref, xo_ref, sv_ref, cv_ref):
    i = pl.program_id(0)
    v = vidx_ref[0, 0, :]
    oh = (v[:, None] == lax.broadcasted_iota(jnp.int32, (BLK_N, B), 1)
          ).astype(jnp.float32)
    agg = s0_ref[...] + s1_ref[...]
    h = pn_ref[...] + _dot(agg, wne[...]) + _dot(oh, un_ref[...])
    h = jnp.maximum(h, 0.0)
    h2 = jnp.maximum(_dot(h, nw2[...]) + nb2_ref[...], 0.0)
    mu = jnp.mean(h2, axis=1, keepdims=True)
    d = h2 - mu
    var = jnp.mean(d * d, axis=1, keepdims=True)
    xo = d * lax.rsqrt(var + 1e-5) * ng_ref[...] + nbeta_ref[...]
    xo_ref[...] = xo

    @pl.when(i == 0)
    def _():
        sv_ref[...] = jnp.zeros_like(sv_ref)
        cv_ref[...] = jnp.zeros_like(cv_ref)

    ohT = (v[None, :] == lax.broadcasted_iota(jnp.int32, (B, BLK_N), 0)
           ).astype(jnp.float32)
    sv_ref[...] += _dot(ohT, xo)
    cv_ref[...] += jnp.broadcast_to(jnp.sum(oh, axis=0)[:, None], (B, 8))


def _node_call(pn, s0, s1, vidx3, un, wne, nw2, nb2, ng, nbeta):
    full = lambda s: pl.BlockSpec(s, lambda i: (0,) * len(s))
    return pl.pallas_call(
        _node_body,
        grid=(GRID_N,),
        in_specs=[
            pl.BlockSpec((BLK_N, H), lambda i: (i, 0)),
            pl.BlockSpec((BLK_N, 16), lambda i: (i, 0)),
            pl.BlockSpec((BLK_N, 16), lambda i: (i, 0)),
            pl.BlockSpec((1, 1, BLK_N), lambda i: (i, 0, 0)),
            full((B, H)), full((16, H)), full((H, 128)), full((1, 128)),
            full((1, 128)), full((1, 128)),
        ],
        out_specs=[
            pl.BlockSpec((BLK_N, 128), lambda i: (i, 0)),
            full((B, 128)), full((B, 8)),
        ],
        out_shape=[
            jax.ShapeDtypeStruct((N, 128), jnp.float32),
            jax.ShapeDtypeStruct((B, 128), jnp.float32),
            jax.ShapeDtypeStruct((B, 8), jnp.float32),
        ],
    )(pn, s0, s1, vidx3, un, wne, nw2, nb2, ng, nbeta)


def _global_body(u_ref, sv_ref, cv_ref, se_ref, ce_ref, gwu, gwv, gwe,
                 gb1_ref, gw2, gb2_ref, gg_ref, gbeta_ref, uo_ref):
    aggv = sv_ref[...] / jnp.maximum(cv_ref[:, 0:1], 1.0)
    aggeT = se_ref[...] / jnp.maximum(ce_ref[0:1, :], 1.0)
    h = (_dot(u_ref[...], gwu[...]) + _dot(aggv, gwv[...])
         + lax.dot_general(aggeT, gwe[...], (((0,), (0,)), ((), ())),
                           preferred_element_type=jnp.float32)
         + gb1_ref[...])
    h = jnp.maximum(h, 0.0)
    h2 = jnp.maximum(_dot(h, gw2[...]) + gb2_ref[...], 0.0)
    mu = jnp.mean(h2, axis=1, keepdims=True)
    d = h2 - mu
    var = jnp.mean(d * d, axis=1, keepdims=True)
    uo_ref[...] = d * lax.rsqrt(var + 1e-5) * gg_ref[...] + gbeta_ref[...]


def _global_call(u, sv, cv, se, ce, gwu, gwv, gwe, gb1, gw2, gb2, gg, gbeta):
    return pl.pallas_call(
        _global_body,
        out_shape=jax.ShapeDtypeStruct((B, 32), jnp.float32),
    )(u, sv, cv, se, ce, gwu, gwv, gwe, gb1, gw2, gb2, gg, gbeta)


def kernel(x, edge_index, edge_attr, u, v_indices, e_indices,
           eW1, eb1, eW2, eb2, eg, ebeta,
           nW1, nb1, nW2, nb2, ng, nbeta,
           gW1, gb1, gW2, gb2, gg, gbeta):
    row = edge_index[0]
    col = edge_index[1]

    wxr, wxc, wea, weu = eW1[:128], eW1[128:256], eW1[256:272], eW1[272:304]
    wnx, wne, wnu = nW1[:128], nW1[128:144], nW1[144:176]
    gwu, gwv, gwe = gW1[:32], gW1[32:160], gW1[160:176]

    r2 = lambda a: a.reshape(1, -1)

    pr, pc, pn, ue, un = _prep_call(x, wxr, wxc, wnx, u, weu, r2(eb1),
                                    wnu, r2(nb1))

    pad = jnp.zeros(((NPADC - NCH) * CH,), jnp.int32)
    rowp = jnp.concatenate([row[:HALF], pad, row[HALF:], pad]
                           ).reshape(2 * NPADC, CH)
    colp = jnp.concatenate([col[:HALF], pad, col[HALF:], pad]
                           ).reshape(2 * NPADC, CH)

    g2 = _gather_call(pr, pc, rowp, colp)

    r2c = lambda a: a.reshape(-1, 1)
    eaT2 = jnp.concatenate([edge_attr[:HALF].T, edge_attr[HALF:].T], axis=0)
    el3 = e_indices[:HALF].reshape(GRID_H, 1, HB)
    eh3 = e_indices[HALF:].reshape(GRID_H, 1, HB)
    zb = jnp.zeros((B, H), jnp.float32)
    uebd = jnp.concatenate(
        [jnp.concatenate([ue, zb], axis=1),
         jnp.concatenate([zb, ue], axis=1)], axis=0)
    zw = jnp.zeros_like(wea)
    weabd = jnp.concatenate(
        [jnp.concatenate([wea, zw], axis=1),
         jnp.concatenate([zw, wea], axis=1)], axis=0)
    zw2 = jnp.zeros_like(eW2)
    w2bd = jnp.concatenate(
        [jnp.concatenate([eW2, zw2], axis=1),
         jnp.concatenate([zw2, eW2], axis=1)], axis=0)
    eoTl, eoTh, se, ce = _edge_call(g2, eaT2, el3, eh3, uebd, weabd, w2bd,
                                    r2c(eb2), r2c(eg), r2c(ebeta))
    eo = jnp.concatenate([eoTl, eoTh], axis=1).T

    parts = _scatter_call(eoTl, eoTh, colp)

    vidx3 = v_indices.reshape(GRID_N, 1, BLK_N)
    xo, sv, cv = _node_call(pn, parts[0], parts[1], vidx3, un, wne, nW2,
                            r2(nb2), r2(ng), r2(nbeta))

    uo = _global_call(u, sv, cv, se, ce, gwu, gwv, gwe, r2(gb1), gW2,
                      r2(gb2), r2(gg), r2(gbeta))
    return (xo, eo, uo)


# speedup vs baseline: 1.0041x; 1.0041x over previous
import functools

import jax
import jax.numpy as jnp
from jax import lax
from jax.experimental import pallas as pl
from jax.experimental.pallas import tpu as pltpu
from jax.experimental.pallas import tpu_sc as plsc

N = 10000
E = 320000
B = 16
H = 64

NC, NS = 2, 16
NW = NC * NS
CH = 128
HALF = E // 2
NCH = HALF // CH
CPW = 40
NPADC = CPW * NW
NPAD_N = 10240
NPT = NPAD_N // NS

HB = 3200
GRID_E = E // HB
GRID_H = HALF // HB
BLK_N = 2000
GRID_N = N // BLK_N

@functools.cache
def _sc_mesh():
    return plsc.VectorSubcoreMesh(core_axis_name="c", subcore_axis_name="s",
                                  num_cores=NC, num_subcores=NS)


def _dot(a, b):
    return jnp.dot(a, b, preferred_element_type=jnp.float32)


def _gather_body(pr, pc, rowp, colp, g2, idxr, idxc, balo, bblo, bahi, bbhi,
                 balo2, bblo2, bahi2, bbhi2, obufa, obufb,
                 sema, semb, semoa, semob):
    w = lax.axis_index("s") * NC + lax.axis_index("c")
    start = w * CPW
    n = jnp.minimum(CPW, NCH - start)
    pltpu.sync_copy(rowp.at[pl.ds(start, CPW)], idxr.at[pl.ds(0, CPW)])
    pltpu.sync_copy(rowp.at[pl.ds(NPADC + start, CPW)],
                    idxr.at[pl.ds(CPW, CPW)])
    pltpu.sync_copy(colp.at[pl.ds(start, CPW)], idxc.at[pl.ds(0, CPW)])
    pltpu.sync_copy(colp.at[pl.ds(NPADC + start, CPW)],
                    idxc.at[pl.ds(CPW, CPW)])

    def issue(j, bufs, sem_g):
        return [pltpu.async_copy(pr.at[idxr.at[j]], bufs[0], sem_g),
                pltpu.async_copy(pc.at[idxc.at[j]], bufs[1], sem_g),
                pltpu.async_copy(pr.at[idxr.at[CPW + j]], bufs[2], sem_g),
                pltpu.async_copy(pc.at[idxc.at[CPW + j]], bufs[3], sem_g)]

    def pack_and_store(j, bufs, sem_g, obuf, sem_o, first):
        pltpu.make_async_copy(pr.at[idxr.at[j]], bufs[0], sem_g).wait()
        pltpu.make_async_copy(pc.at[idxc.at[j]], bufs[1], sem_g).wait()
        pltpu.make_async_copy(pr.at[idxr.at[CPW + j]], bufs[2], sem_g).wait()
        pltpu.make_async_copy(pc.at[idxc.at[CPW + j]], bufs[3], sem_g).wait()

        @pl.when(jnp.logical_not(first))
        def _():
            pltpu.make_async_copy(obuf, g2.at[pl.ds(0, CH)], sem_o).wait()

        def packrow(p, carry2):
            for r in range(4):
                i = 4 * p + r
                for k in range(H // 16):
                    sl = pl.ds(k * 16, 16)
                    obuf[i, pl.ds(k * 16, 16)] = bufs[0][i, sl] + bufs[1][i, sl]
                    obuf[i, pl.ds(H + k * 16, 16)] = (bufs[2][i, sl]
                                                      + bufs[3][i, sl])
            return carry2

        lax.fori_loop(0, CH // 4, packrow, 0)
        pltpu.async_copy(obuf, g2.at[pl.ds((start + j) * CH, CH)], sem_o)

    seta = (balo, bblo, bahi, bbhi)
    setb = (balo2, bblo2, bahi2, bbhi2)

    issue(0, seta, sema)

    def pair(t, carry):
        j = 2 * t

        @pl.when(j + 1 < n)
        def _():
            issue(j + 1, setb, semb)

        pack_and_store(j, seta, sema, obufa, semoa, t == 0)

        @pl.when(j + 1 < n)
        def _():
            @pl.when(j + 2 < n)
            def _():
                issue(j + 2, seta, sema)

            pack_and_store(j + 1, setb, semb, obufb, semob, t == 0)

        return carry

    lax.fori_loop(0, (n + 1) // 2, pair, 0)
    pltpu.make_async_copy(obufa, g2.at[pl.ds(0, CH)], semoa).wait()
    pltpu.make_async_copy(obufb, g2.at[pl.ds(0, CH)], semob).wait()


@functools.cache
def _gather_kernel():
    return pl.kernel(
        _gather_body,
        out_type=jax.ShapeDtypeStruct((HALF, 128), jnp.float32),
        mesh=_sc_mesh(),
        compiler_params=pltpu.CompilerParams(use_tc_tiling_on_sc=False, needs_layout_passes=False),
        scratch_types=[
            pltpu.VMEM((2 * CPW, CH), jnp.int32),
            pltpu.VMEM((2 * CPW, CH), jnp.int32),
            pltpu.VMEM((CH, H), jnp.float32),
            pltpu.VMEM((CH, H), jnp.float32),
            pltpu.VMEM((CH, H), jnp.float32),
            pltpu.VMEM((CH, H), jnp.float32),
            pltpu.VMEM((CH, H), jnp.float32),
            pltpu.VMEM((CH, H), jnp.float32),
            pltpu.VMEM((CH, H), jnp.float32),
            pltpu.VMEM((CH, H), jnp.float32),
            pltpu.VMEM((CH, 128), jnp.float32),
            pltpu.VMEM((CH, 128), jnp.float32),
            pltpu.SemaphoreType.DMA,
            pltpu.SemaphoreType.DMA,
            pltpu.SemaphoreType.DMA,
            pltpu.SemaphoreType.DMA,
        ],
    )


def _gather_call(pr, pc, rowp, colp):
    return _gather_kernel()(pr, pc, rowp, colp)


def _scatter_body(eoTl, eoTh, colp, parts, idxc, dbufT, dbufT2, dbuf, dbuf2,
                  zbuf, shared, semil, semih, semal, semah):
    c_ax = lax.axis_index("c")
    s_ax = lax.axis_index("s")
    w = s_ax * NC + c_ax
    start = w * CPW
    n = jnp.minimum(CPW, NCH - start)
    pltpu.sync_copy(colp.at[pl.ds(start, CPW)], idxc.at[pl.ds(0, CPW)])
    pltpu.sync_copy(colp.at[pl.ds(NPADC + start, CPW)],
                    idxc.at[pl.ds(CPW, CPW)])

    def zrow(i, carry):
        zbuf[i, :] = jnp.zeros((16,), jnp.float32)
        return carry

    lax.fori_loop(0, NPT, zrow, 0)
    pltpu.sync_copy(zbuf, shared.at[pl.ds(s_ax * NPT, NPT)])
    plsc.subcore_barrier()

    lane = lax.broadcasted_iota(jnp.int32, (16,), 0)

    def transpose(dT, db):
        for g in range(CH // 16):
            idx_e = lane + (g * 16)
            for f in range(16):
                plsc.store_scatter(db, [idx_e, jnp.full((16,), f, jnp.int32)],
                                   dT[f, pl.ds(g * 16, 16)])

    def half(k, src, dT, db, sem_i, sem_a, idx_row, prefetch, first):
        pltpu.make_async_copy(src.at[:, pl.ds(0, CH)], dT, sem_i).wait()

        @pl.when(jnp.logical_not(first))
        def _():
            pltpu.make_async_copy(db, shared.at[idxc.at[0]], sem_a).wait()

        transpose(dT, db)
        pltpu.async_copy(db, shared.at[idx_row], sem_a, add=True)

        @pl.when(k + 1 < n)
        def _():
            prefetch(k + 1)

    def pre_lo(k):
        pltpu.async_copy(eoTl.at[:, pl.ds((start + k) * CH, CH)], dbufT, semil)

    def pre_hi(k):
        pltpu.async_copy(eoTh.at[:, pl.ds((start + k) * CH, CH)], dbufT2,
                         semih)

    pre_lo(0)
    pre_hi(0)

    def chunk(j, carry):
        half(j, eoTl, dbufT, dbuf, semil, semal, idxc.at[j], pre_lo, j == 0)
        half(j, eoTh, dbufT2, dbuf2, semih, semah, idxc.at[CPW + j], pre_hi,
             j == 0)
        return carry

    lax.fori_loop(0, n, chunk, 0)
    pltpu.make_async_copy(dbuf, shared.at[idxc.at[0]], semal).wait()
    pltpu.make_async_copy(dbuf2, shared.at[idxc.at[0]], semah).wait()
    plsc.subcore_barrier()
    pltpu.sync_copy(shared.at[pl.ds(s_ax * NPT, NPT)],
                    parts.at[c_ax, pl.ds(s_ax * NPT, NPT)])


@functools.cache
def _scatter_kernel():
    return pl.kernel(
        _scatter_body,
        out_type=jax.ShapeDtypeStruct((NC, NPAD_N, 16), jnp.float32),
        mesh=_sc_mesh(),
        compiler_params=pltpu.CompilerParams(use_tc_tiling_on_sc=False, needs_layout_passes=False),
        scratch_types=[
            pltpu.VMEM((2 * CPW, CH), jnp.int32),
            pltpu.VMEM((16, CH), jnp.float32),
            pltpu.VMEM((16, CH), jnp.float32),
            pltpu.VMEM((CH, 16), jnp.float32),
            pltpu.VMEM((CH, 16), jnp.float32),
            pltpu.VMEM((NPT, 16), jnp.float32),
            pltpu.VMEM_SHARED((NPAD_N, 16), jnp.float32),
            pltpu.SemaphoreType.DMA,
            pltpu.SemaphoreType.DMA,
            pltpu.SemaphoreType.DMA,
            pltpu.SemaphoreType.DMA,
        ],
    )


def _scatter_call(eoTl, eoTh, colp):
    return _scatter_kernel()(eoTl, eoTh, colp)


def _prep_body(x_ref, wxr, wxc, wnx, u_ref, weu, eb1_ref, wnu, nb1_ref,
               pr_ref, pc_ref, pn_ref, ue_ref, un_ref):
    xb = x_ref[...]
    pr_ref[...] = _dot(xb, wxr[...])
    pc_ref[...] = _dot(xb, wxc[...])
    pn_ref[...] = _dot(xb, wnx[...])
    ub = u_ref[...]
    ue_ref[...] = _dot(ub, weu[...]) + eb1_ref[...]
    un_ref[...] = _dot(ub, wnu[...]) + nb1_ref[...]


def _prep_call(x, wxr, wxc, wnx, u, weu, eb1, wnu, nb1):
    full = lambda s: pl.BlockSpec(s, lambda i: (0, 0))
    return pl.pallas_call(
        _prep_body,
        grid=(GRID_N,),
        in_specs=[
            pl.BlockSpec((BLK_N, 128), lambda i: (i, 0)),
            full((128, H)), full((128, H)), full((128, H)),
            full((B, 32)), full((32, H)), full((1, H)), full((32, H)),
            full((1, H)),
        ],
        out_specs=[
            pl.BlockSpec((BLK_N, H), lambda i: (i, 0)),
            pl.BlockSpec((BLK_N, H), lambda i: (i, 0)),
            pl.BlockSpec((BLK_N, H), lambda i: (i, 0)),
            full((B, H)), full((B, H)),
        ],
        out_shape=[
            jax.ShapeDtypeStruct((N, H), jnp.float32),
            jax.ShapeDtypeStruct((N, H), jnp.float32),
            jax.ShapeDtypeStruct((N, H), jnp.float32),
            jax.ShapeDtypeStruct((B, H), jnp.float32),
            jax.ShapeDtypeStruct((B, H), jnp.float32),
        ],
    )(x, wxr, wxc, wnx, u, weu, eb1, wnu, nb1)


def _ln_T(h2, eb2c, egc, ebetac):
    h2 = jnp.maximum(h2 + eb2c, 0.0)
    mu = jnp.mean(h2, axis=0, keepdims=True)
    d = h2 - mu
    var = jnp.mean(d * d, axis=0, keepdims=True)
    return d * lax.rsqrt(var + 1e-5) * egc + ebetac


def _edge_body(g2_ref, eaT2_ref, el_ref, eh_ref, uebd_ref, weabd, w2bd,
               eb2_ref, eg_ref, ebeta_ref, eoTl_ref, eoTh_ref, se_ref,
               ce_ref):
    i = pl.program_id(0)
    dg = lambda a, b, dims: lax.dot_general(a, b, (dims, ((), ())),
                                            preferred_element_type=jnp.float32)
    el = el_ref[0, 0, :][None, :]
    eh = eh_ref[0, 0, :][None, :]
    gid = lax.broadcasted_iota(jnp.int32, (B, HB), 0)
    ohl = (el == gid).astype(jnp.float32)
    ohh = (eh == gid).astype(jnp.float32)
    oh2 = jnp.concatenate([ohl, ohh], axis=0)
    a = dg(eaT2_ref[...], weabd[...], ((0,), (0,)))
    u = dg(oh2, uebd_ref[...], ((0,), (0,)))
    h = jnp.maximum(g2_ref[...] + a + u, 0.0)
    h2 = dg(w2bd[...], h, ((0,), (1,)))
    eoTl = _ln_T(h2[:16], eb2_ref[...], eg_ref[...], ebeta_ref[...])
    eoTh = _ln_T(h2[16:], eb2_ref[...], eg_ref[...], ebeta_ref[...])
    eoTl_ref[...] = eoTl
    eoTh_ref[...] = eoTh

    @pl.when(i == 0)
    def _():
        se_ref[...] = jnp.zeros_like(se_ref)
        ce_ref[...] = jnp.zeros_like(ce_ref)

    se_ref[...] += (dg(eoTl, ohl, ((1,), (1,)))
                    + dg(eoTh, ohh, ((1,), (1,))))
    cnt_col = (jnp.sum(ohl, axis=1, keepdims=True)
               + jnp.sum(ohh, axis=1, keepdims=True))
    eye = (lax.broadcasted_iota(jnp.int32, (B, B), 0)
           == lax.broadcasted_iota(jnp.int32, (B, B), 1)).astype(jnp.float32)
    cnt_row = dg(cnt_col, eye, ((0,), (0,)))
    ce_ref[...] += jnp.broadcast_to(cnt_row, (8, B))


def _edge_call(g2, eaT2, el3, eh3, uebd, weabd, w2bd, eb2c, egc, ebetac):
    full = lambda s: pl.BlockSpec(s, lambda i: (0,) * len(s))
    return pl.pallas_call(
        _edge_body,
        grid=(GRID_H,),
        in_specs=[
            pl.BlockSpec((HB, 128), lambda i: (i, 0)),
            pl.BlockSpec((32, HB), lambda i: (0, i)),
            pl.BlockSpec((1, 1, HB), lambda i: (i, 0, 0)),
            pl.BlockSpec((1, 1, HB), lambda i: (i, 0, 0)),
            full((2 * B, 128)), full((32, 128)), full((128, 32)),
            full((16, 1)), full((16, 1)), full((16, 1)),
        ],
        out_specs=[
            pl.BlockSpec((16, HB), lambda i: (0, i)),
            pl.BlockSpec((16, HB), lambda i: (0, i)),
            full((16, B)), full((8, B)),
        ],
        out_shape=[
            jax.ShapeDtypeStruct((16, HALF), jnp.float32),
            jax.ShapeDtypeStruct((16, HALF), jnp.float32),
            jax.ShapeDtypeStruct((16, B), jnp.float32),
            jax.ShapeDtypeStruct((8, B), jnp.float32),
        ],
    )(g2, eaT2, el3, eh3, uebd, weabd, w2bd, eb2c, egc, ebetac)


def _node_body(pn_ref, s0_ref, s1_ref, vidx_ref, un_ref, wne, nw2, nb2_ref,
               ng_ref, nbeta_ref, xo_ref, sv_ref, cv_ref):
    i = pl.program_id(0)
    v = vidx_ref[0, 0, :]
    oh = (v[:, None] == lax.broadcasted_iota(jnp.int32, (BLK_N, B), 1)
          ).astype(jnp.float32)
    agg = s0_ref[...] + s1_ref[...]
    h = pn_ref[...] + _dot(agg, wne[...]) + _dot(oh, un_ref[...])
    h = jnp.maximum(h, 0.0)
    h2 = jnp.maximum(_dot(h, nw2[...]) + nb2_ref[...], 0.0)
    mu = jnp.mean(h2, axis=1, keepdims=True)
    d = h2 - mu
    var = jnp.mean(d * d, axis=1, keepdims=True)
    xo = d * lax.rsqrt(var + 1e-5) * ng_ref[...] + nbeta_ref[...]
    xo_ref[...] = xo

    @pl.when(i == 0)
    def _():
        sv_ref[...] = jnp.zeros_like(sv_ref)
        cv_ref[...] = jnp.zeros_like(cv_ref)

    ohT = (v[None, :] == lax.broadcasted_iota(jnp.int32, (B, BLK_N), 0)
           ).astype(jnp.float32)
    sv_ref[...] += _dot(ohT, xo)
    cv_ref[...] += jnp.broadcast_to(jnp.sum(oh, axis=0)[:, None], (B, 8))


def _node_call(pn, s0, s1, vidx3, un, wne, nw2, nb2, ng, nbeta):
    full = lambda s: pl.BlockSpec(s, lambda i: (0,) * len(s))
    return pl.pallas_call(
        _node_body,
        grid=(GRID_N,),
        in_specs=[
            pl.BlockSpec((BLK_N, H), lambda i: (i, 0)),
            pl.BlockSpec((BLK_N, 16), lambda i: (i, 0)),
            pl.BlockSpec((BLK_N, 16), lambda i: (i, 0)),
            pl.BlockSpec((1, 1, BLK_N), lambda i: (i, 0, 0)),
            full((B, H)), full((16, H)), full((H, 128)), full((1, 128)),
            full((1, 128)), full((1, 128)),
        ],
        out_specs=[
            pl.BlockSpec((BLK_N, 128), lambda i: (i, 0)),
            full((B, 128)), full((B, 8)),
        ],
        out_shape=[
            jax.ShapeDtypeStruct((N, 128), jnp.float32),
            jax.ShapeDtypeStruct((B, 128), jnp.float32),
            jax.ShapeDtypeStruct((B, 8), jnp.float32),
        ],
    )(pn, s0, s1, vidx3, un, wne, nw2, nb2, ng, nbeta)


def _global_body(u_ref, sv_ref, cv_ref, se_ref, ce_ref, gwu, gwv, gwe,
                 gb1_ref, gw2, gb2_ref, gg_ref, gbeta_ref, uo_ref):
    aggv = sv_ref[...] / jnp.maximum(cv_ref[:, 0:1], 1.0)
    aggeT = se_ref[...] / jnp.maximum(ce_ref[0:1, :], 1.0)
    h = (_dot(u_ref[...], gwu[...]) + _dot(aggv, gwv[...])
         + lax.dot_general(aggeT, gwe[...], (((0,), (0,)), ((), ())),
                           preferred_element_type=jnp.float32)
         + gb1_ref[...])
    h = jnp.maximum(h, 0.0)
    h2 = jnp.maximum(_dot(h, gw2[...]) + gb2_ref[...], 0.0)
    mu = jnp.mean(h2, axis=1, keepdims=True)
    d = h2 - mu
    var = jnp.mean(d * d, axis=1, keepdims=True)
    uo_ref[...] = d * lax.rsqrt(var + 1e-5) * gg_ref[...] + gbeta_ref[...]


def _global_call(u, sv, cv, se, ce, gwu, gwv, gwe, gb1, gw2, gb2, gg, gbeta):
    return pl.pallas_call(
        _global_body,
        out_shape=jax.ShapeDtypeStruct((B, 32), jnp.float32),
    )(u, sv, cv, se, ce, gwu, gwv, gwe, gb1, gw2, gb2, gg, gbeta)


def kernel(x, edge_index, edge_attr, u, v_indices, e_indices,
           eW1, eb1, eW2, eb2, eg, ebeta,
           nW1, nb1, nW2, nb2, ng, nbeta,
           gW1, gb1, gW2, gb2, gg, gbeta):
    row = edge_index[0]
    col = edge_index[1]

    wxr, wxc, wea, weu = eW1[:128], eW1[128:256], eW1[256:272], eW1[272:304]
    wnx, wne, wnu = nW1[:128], nW1[128:144], nW1[144:176]
    gwu, gwv, gwe = gW1[:32], gW1[32:160], gW1[160:176]

    r2 = lambda a: a.reshape(1, -1)

    pr, pc, pn, ue, un = _prep_call(x, wxr, wxc, wnx, u, weu, r2(eb1),
                                    wnu, r2(nb1))

    pad = jnp.zeros(((NPADC - NCH) * CH,), jnp.int32)
    rowp = jnp.concatenate([row[:HALF], pad, row[HALF:], pad]
                           ).reshape(2 * NPADC, CH)
    colp = jnp.concatenate([col[:HALF], pad, col[HALF:], pad]
                           ).reshape(2 * NPADC, CH)

    g2 = _gather_call(pr, pc, rowp, colp)

    r2c = lambda a: a.reshape(-1, 1)
    eaT2 = jnp.concatenate([edge_attr[:HALF].T, edge_attr[HALF:].T], axis=0)
    el3 = e_indices[:HALF].reshape(GRID_H, 1, HB)
    eh3 = e_indices[HALF:].reshape(GRID_H, 1, HB)
    zb = jnp.zeros((B, H), jnp.float32)
    uebd = jnp.concatenate(
        [jnp.concatenate([ue, zb], axis=1),
         jnp.concatenate([zb, ue], axis=1)], axis=0)
    zw = jnp.zeros_like(wea)
    weabd = jnp.concatenate(
        [jnp.concatenate([wea, zw], axis=1),
         jnp.concatenate([zw, wea], axis=1)], axis=0)
    zw2 = jnp.zeros_like(eW2)
    w2bd = jnp.concatenate(
        [jnp.concatenate([eW2, zw2], axis=1),
         jnp.concatenate([zw2, eW2], axis=1)], axis=0)
    eoTl, eoTh, se, ce = _edge_call(g2, eaT2, el3, eh3, uebd, weabd, w2bd,
                                    r2c(eb2), r2c(eg), r2c(ebeta))
    eo = jnp.concatenate([eoTl, eoTh], axis=1).T

    parts = _scatter_call(eoTl, eoTh, colp)

    vidx3 = v_indices.reshape(GRID_N, 1, BLK_N)
    xo, sv, cv = _node_call(pn, parts[0], parts[1], vidx3, un, wne, nW2,
                            r2(nb2), r2(ng), r2(nbeta))

    uo = _global_call(u, sv, cv, se, ce, gwu, gwv, gwe, r2(gb1), gW2,
                      r2(gb2), r2(gg), r2(gbeta))
    return (xo, eo, uo)
